# baseline (speedup 1.0000x reference)
import jax
import jax.numpy as jnp
from jax import lax
from jax.experimental import pallas as pl
from jax.experimental.pallas import tpu as pltpu
from jax.experimental.pallas import tpu_sc as plsc

B = 1024
D = 512
NUM_CLASS = 100000
ROW = 77 * D
OFF_CLS = 5 * D
OFF_MID = 9 * D
OFF_CLOTH = 11 * D
OFF_SUF = 15 * D

NW = 32
BPW = B // NW


def _body(label_ref, cls_ref, cloth_ref, pre_ref, mid_ref, suf_ref, out_ref,
          idx_v, g_v, pre_v, mid_v, suf_v, sem):
    wid = lax.axis_index("s") * 2 + lax.axis_index("c")
    base = wid * BPW

    pltpu.sync_copy(label_ref.at[pl.ds(base, BPW)], idx_v)
    pltpu.sync_copy(pre_ref, pre_v)
    pltpu.sync_copy(mid_ref, mid_v)
    pltpu.sync_copy(suf_ref, suf_v)

    pltpu.async_copy(cls_ref.at[idx_v], g_v, sem).wait()

    @pl.loop(0, BPW)
    def _(i):
        pltpu.sync_copy(g_v.at[i], out_ref.at[base + i, pl.ds(OFF_CLS, 4 * D)])

    pltpu.async_copy(cloth_ref.at[idx_v], g_v, sem).wait()

    @pl.loop(0, BPW)
    def _(i):
        pltpu.sync_copy(g_v.at[i], out_ref.at[base + i, pl.ds(OFF_CLOTH, 4 * D)])
        pltpu.sync_copy(pre_v, out_ref.at[base + i, pl.ds(0, 5 * D)])
        pltpu.sync_copy(mid_v, out_ref.at[base + i, pl.ds(OFF_MID, 2 * D)])
        pltpu.sync_copy(suf_v, out_ref.at[base + i, pl.ds(OFF_SUF, 62 * D)])


def kernel(label, cls_ctx, cloth_cls_ctx, token_prefix, token_mid, token_suffix):
    cls2 = cls_ctx.reshape(NUM_CLASS, 4 * D)
    cloth2 = cloth_cls_ctx.reshape(NUM_CLASS, 4 * D)
    pre = token_prefix.reshape(5 * D)
    mid = token_mid.reshape(2 * D)
    suf = token_suffix.reshape(62 * D)
    lbl = label.astype(jnp.int32)

    mesh = plsc.VectorSubcoreMesh(core_axis_name="c", subcore_axis_name="s")
    out = pl.kernel(
        _body,
        out_type=jax.ShapeDtypeStruct((B, ROW), jnp.float32),
        mesh=mesh,
        scratch_types=[
            pltpu.VMEM((BPW,), jnp.int32),
            pltpu.VMEM((BPW, 4 * D), jnp.float32),
            pltpu.VMEM((5 * D,), jnp.float32),
            pltpu.VMEM((2 * D,), jnp.float32),
            pltpu.VMEM((62 * D,), jnp.float32),
            pltpu.SemaphoreType.DMA,
        ],
    )(lbl, cls2, cloth2, pre, mid, suf)
    return out.reshape(B, 77, D)


# speedup vs baseline: 1.0005x; 1.0005x over previous
import jax
import jax.numpy as jnp
from jax import lax
from jax.experimental import pallas as pl
from jax.experimental.pallas import tpu as pltpu
from jax.experimental.pallas import tpu_sc as plsc

B = 1024
D = 512
NUM_CLASS = 100000
ROW = 77 * D
OFF_CLS = 5 * D
OFF_MID = 9 * D
OFF_CLOTH = 11 * D
OFF_SUF = 15 * D

NW = 32
BPW = B // NW


def _body(label_ref, cls_ref, cloth_ref, pre_ref, mid_ref, suf_ref, out_ref,
          idx_v, g_v, pre_v, mid_v, suf_v,
          sem_g, sem_w, sem_pre, sem_mid, sem_suf):
    wid = lax.axis_index("s") * 2 + lax.axis_index("c")
    base = wid * BPW

    pltpu.sync_copy(label_ref.at[pl.ds(base, BPW)], idx_v)
    pltpu.sync_copy(pre_ref, pre_v)
    pltpu.sync_copy(mid_ref, mid_v)
    pltpu.sync_copy(suf_ref, suf_v)

    pltpu.async_copy(cls_ref.at[idx_v], g_v, sem_g).wait()

    @pl.loop(0, BPW)
    def _(i):
        pltpu.async_copy(g_v.at[i], out_ref.at[base + i, pl.ds(OFF_CLS, 4 * D)],
                         sem_w)

    @pl.loop(0, BPW)
    def _(i):
        pltpu.async_copy(pre_v, out_ref.at[base + i, pl.ds(0, 5 * D)], sem_pre)
        pltpu.async_copy(mid_v, out_ref.at[base + i, pl.ds(OFF_MID, 2 * D)],
                         sem_mid)
        pltpu.async_copy(suf_v, out_ref.at[base + i, pl.ds(OFF_SUF, 62 * D)],
                         sem_suf)

    @pl.loop(0, BPW)
    def _(i):
        pltpu.make_async_copy(g_v.at[i],
                              out_ref.at[base + i, pl.ds(OFF_CLS, 4 * D)],
                              sem_w).wait()

    pltpu.async_copy(cloth_ref.at[idx_v], g_v, sem_g).wait()

    @pl.loop(0, BPW)
    def _(i):
        pltpu.async_copy(g_v.at[i],
                         out_ref.at[base + i, pl.ds(OFF_CLOTH, 4 * D)], sem_w)

    @pl.loop(0, BPW)
    def _(i):
        pltpu.make_async_copy(g_v.at[i],
                              out_ref.at[base + i, pl.ds(OFF_CLOTH, 4 * D)],
                              sem_w).wait()
        pltpu.make_async_copy(pre_v, out_ref.at[base + i, pl.ds(0, 5 * D)],
                              sem_pre).wait()
        pltpu.make_async_copy(mid_v, out_ref.at[base + i, pl.ds(OFF_MID, 2 * D)],
                              sem_mid).wait()
        pltpu.make_async_copy(suf_v, out_ref.at[base + i, pl.ds(OFF_SUF, 62 * D)],
                              sem_suf).wait()


def kernel(label, cls_ctx, cloth_cls_ctx, token_prefix, token_mid, token_suffix):
    cls2 = cls_ctx.reshape(NUM_CLASS, 4 * D)
    cloth2 = cloth_cls_ctx.reshape(NUM_CLASS, 4 * D)
    pre = token_prefix.reshape(5 * D)
    mid = token_mid.reshape(2 * D)
    suf = token_suffix.reshape(62 * D)
    lbl = label.astype(jnp.int32)

    mesh = plsc.VectorSubcoreMesh(core_axis_name="c", subcore_axis_name="s")
    out = pl.kernel(
        _body,
        out_type=jax.ShapeDtypeStruct((B, ROW), jnp.float32),
        mesh=mesh,
        scratch_types=[
            pltpu.VMEM((BPW,), jnp.int32),
            pltpu.VMEM((BPW, 4 * D), jnp.float32),
            pltpu.VMEM((5 * D,), jnp.float32),
            pltpu.VMEM((2 * D,), jnp.float32),
            pltpu.VMEM((62 * D,), jnp.float32),
            pltpu.SemaphoreType.DMA,
            pltpu.SemaphoreType.DMA,
            pltpu.SemaphoreType.DMA,
            pltpu.SemaphoreType.DMA,
            pltpu.SemaphoreType.DMA,
        ],
    )(lbl, cls2, cloth2, pre, mid, suf)
    return out.reshape(B, 77, D)


# speedup vs baseline: 6.8505x; 6.8469x over previous
import jax
import jax.numpy as jnp
from jax import lax
from jax.experimental import pallas as pl
from jax.experimental.pallas import tpu as pltpu
from jax.experimental.pallas import tpu_sc as plsc

B = 1024
D = 512
NUM_CLASS = 100000

NW = 32
BPW = B // NW
CH = 8
NCH = BPW // CH
WV = CH // 2


def _sc_body(label_ref, cls_ref, cloth_ref, tmpl_ref, tail_ref, out_ref,
             idx_v, g_v, hd_v, tail_v, sem_g, sem_h, sem_s, sem_l):
    wid = lax.axis_index("s") * 2 + lax.axis_index("c")
    base = wid * BPW

    pltpu.sync_copy(label_ref.at[pl.ds(base, BPW)], idx_v)
    pltpu.async_copy(tail_ref, tail_v, sem_s)
    for r in range(CH):
        pltpu.async_copy(tmpl_ref, hd_v.at[r], sem_l)
    pltpu.make_async_copy(tail_ref, tail_v, sem_s).wait()
    for r in range(CH):
        pltpu.make_async_copy(tmpl_ref, hd_v.at[r], sem_l).wait()

    @pl.loop(0, NCH)
    def _(c):
        crow = base + c * CH

        @pl.when(c >= 1)
        def _():
            for w in range(2):
                pltpu.make_async_copy(
                    hd_v.at[pl.ds(w * WV, WV)],
                    out_ref.at[pl.ds(crow - CH + w * WV, WV), pl.ds(0, 16)],
                    sem_h).wait()

        pltpu.async_copy(cls_ref.at[idx_v.at[pl.ds(c * CH, CH)]], g_v,
                         sem_g).wait()

        @pl.loop(0, CH)
        def _(r):
            for t in range(4):
                for v in range(D // 16):
                    sl = pl.ds(v * 16, 16)
                    hd_v[r, 5 + t, sl] = g_v[r, t, sl]

        pltpu.async_copy(cloth_ref.at[idx_v.at[pl.ds(c * CH, CH)]], g_v,
                         sem_g).wait()

        @pl.loop(0, CH)
        def _(r):
            for t in range(4):
                for v in range(D // 16):
                    sl = pl.ds(v * 16, 16)
                    hd_v[r, 11 + t, sl] = g_v[r, t, sl]

        for w in range(2):
            pltpu.async_copy(
                hd_v.at[pl.ds(w * WV, WV)],
                out_ref.at[pl.ds(crow + w * WV, WV), pl.ds(0, 16)], sem_h)

        @pl.loop(0, CH)
        def _(r):
            pltpu.async_copy(tail_v, out_ref.at[crow + r, pl.ds(16, 61)],
                             sem_s)

    for w in range(2):
        pltpu.make_async_copy(
            hd_v.at[pl.ds(w * WV, WV)],
            out_ref.at[pl.ds(base + (NCH - 1) * CH + w * WV, WV),
                       pl.ds(0, 16)],
            sem_h).wait()

    @pl.loop(0, BPW)
    def _(i):
        pltpu.make_async_copy(tail_v, out_ref.at[base + i, pl.ds(16, 61)],
                              sem_s).wait()


def kernel(label, cls_ctx, cloth_cls_ctx, token_prefix, token_mid, token_suffix):
    lbl = label.astype(jnp.int32)
    pre = token_prefix.reshape(5, D)
    mid = token_mid.reshape(2, D)
    suf = token_suffix.reshape(62, D)
    tmpl = jnp.concatenate(
        [pre, jnp.zeros((4, D), jnp.float32), mid,
         jnp.zeros((4, D), jnp.float32), suf[:1]], axis=0)
    tail = suf[1:]

    out_ref = jax.new_ref(lax.empty((B, 77, D), jnp.float32))
    mesh = plsc.VectorSubcoreMesh(core_axis_name="c", subcore_axis_name="s")
    pl.kernel(
        _sc_body,
        out_type=(),
        mesh=mesh,
        scratch_types=[
            pltpu.VMEM((BPW,), jnp.int32),
            pltpu.VMEM((CH, 4, D), jnp.float32),
            pltpu.VMEM((CH, 16, D), jnp.float32),
            pltpu.VMEM((61, D), jnp.float32),
            pltpu.SemaphoreType.DMA,
            pltpu.SemaphoreType.DMA,
            pltpu.SemaphoreType.DMA,
            pltpu.SemaphoreType.DMA,
        ],
    )(lbl, cls_ctx, cloth_cls_ctx, tmpl, tail, out_ref)
    return out_ref[...]


# speedup vs baseline: 7.0277x; 1.0259x over previous
import jax
import jax.numpy as jnp
from jax import lax
from jax.experimental import pallas as pl
from jax.experimental.pallas import tpu as pltpu
from jax.experimental.pallas import tpu_sc as plsc

B = 1024
D = 512
NUM_CLASS = 100000

NW = 32
BPW = B // NW
CH = 8
NCH = BPW // CH
WV = CH // 2


def _sc_body(label_ref, cls_ref, cloth_ref, tmpl_ref, tail_ref, out_ref,
             idx_v, ga_v, gb_v, hd_v, tail_v, sem_g, sem_h, sem_s, sem_l):
    wid = lax.axis_index("s") * 2 + lax.axis_index("c")
    base = wid * BPW

    pltpu.sync_copy(label_ref.at[pl.ds(base, BPW)], idx_v)
    @pl.when(lax.axis_index("s") == 0)
    def _():
        pltpu.sync_copy(tail_ref, tail_v)

    for r in range(CH):
        pltpu.async_copy(tmpl_ref, hd_v.at[r], sem_l)
    for r in range(CH):
        pltpu.make_async_copy(tmpl_ref, hd_v.at[r], sem_l).wait()
    plsc.subcore_barrier()

    @pl.loop(0, BPW)
    def _(i):
        pltpu.async_copy(tail_v, out_ref.at[base + i, pl.ds(16, 61)], sem_s)

    @pl.loop(0, NCH)
    def _(c):
        crow = base + c * CH

        g1 = pltpu.async_copy(cls_ref.at[idx_v.at[pl.ds(c * CH, CH)]], ga_v,
                              sem_g)
        g2 = pltpu.async_copy(cloth_ref.at[idx_v.at[pl.ds(c * CH, CH)]], gb_v,
                              sem_g)
        g1.wait()
        g2.wait()

        for w in range(2):
            @pl.when(c >= 1)
            def _():
                pltpu.make_async_copy(
                    hd_v.at[pl.ds(w * WV, WV)],
                    out_ref.at[pl.ds(crow - CH + w * WV, WV), pl.ds(0, 16)],
                    sem_h).wait()

            @pl.loop(0, WV)
            def _(r):
                row = w * WV + r
                for t in range(4):
                    for v in range(D // 16):
                        sl = pl.ds(v * 16, 16)
                        hd_v[row, 5 + t, sl] = ga_v[row, t, sl]
                        hd_v[row, 11 + t, sl] = gb_v[row, t, sl]

            pltpu.async_copy(
                hd_v.at[pl.ds(w * WV, WV)],
                out_ref.at[pl.ds(crow + w * WV, WV), pl.ds(0, 16)], sem_h)

    for w in range(2):
        pltpu.make_async_copy(
            hd_v.at[pl.ds(w * WV, WV)],
            out_ref.at[pl.ds(base + (NCH - 1) * CH + w * WV, WV),
                       pl.ds(0, 16)],
            sem_h).wait()

    @pl.loop(0, BPW)
    def _(i):
        pltpu.make_async_copy(tail_v, out_ref.at[base + i, pl.ds(16, 61)],
                              sem_s).wait()


def kernel(label, cls_ctx, cloth_cls_ctx, token_prefix, token_mid, token_suffix):
    lbl = label.astype(jnp.int32)
    pre = token_prefix.reshape(5, D)
    mid = token_mid.reshape(2, D)
    suf = token_suffix.reshape(62, D)
    tmpl = jnp.concatenate(
        [pre, jnp.zeros((4, D), jnp.float32), mid,
         jnp.zeros((4, D), jnp.float32), suf[:1]], axis=0)
    tail = suf[1:]

    out_ref = jax.new_ref(lax.empty((B, 77, D), jnp.float32))
    mesh = plsc.VectorSubcoreMesh(core_axis_name="c", subcore_axis_name="s")
    pl.kernel(
        _sc_body,
        out_type=(),
        mesh=mesh,
        scratch_types=[
            pltpu.VMEM((BPW,), jnp.int32),
            pltpu.VMEM((CH, 4, D), jnp.float32),
            pltpu.VMEM((CH, 4, D), jnp.float32),
            pltpu.VMEM((CH, 16, D), jnp.float32),
            pltpu.VMEM_SHARED((61, D), jnp.float32),
            pltpu.SemaphoreType.DMA,
            pltpu.SemaphoreType.DMA,
            pltpu.SemaphoreType.DMA,
            pltpu.SemaphoreType.DMA,
        ],
    )(lbl, cls_ctx, cloth_cls_ctx, tmpl, tail, out_ref)
    return out_ref[...]
